# baseline (speedup 1.0000x reference)
import functools

import jax
import jax.numpy as jnp
from jax import lax
from jax.experimental import pallas as pl
from jax.experimental.pallas import tpu as pltpu
from jax.experimental.pallas import tpu_sc as plsc

N = 10000
NP = 10240
E = 320000
F = 128
NCLS = 40
DPAD = 48
DEGW = 16

NC, NS = 2, 16
NW = NC * NS
CH = 128
NCW = 79
NCHT = NW * NCW
EPAD = NCHT * CH - E
RPS = NP // NS


def _vmesh():
    return plsc.VectorSubcoreMesh(core_axis_name="c", subcore_axis_name="s")


_SC_PARAMS = pltpu.CompilerParams(use_tc_tiling_on_sc=False)


def _sc_degree(dst3d):
    ones_h = jnp.ones((CH, DEGW), jnp.float32)
    zeros_h = jnp.zeros((RPS, DEGW), jnp.float32)

    @functools.partial(
        pl.kernel,
        out_type=jax.ShapeDtypeStruct((NC * NP, DEGW), jnp.float32),
        mesh=_vmesh(),
        scratch_types=[
            pltpu.VMEM((NCW, CH), jnp.int32),
            pltpu.VMEM((CH, DEGW), jnp.float32),
            pltpu.VMEM_SHARED((NP, DEGW), jnp.float32),
        ],
    )
    def deg_kernel(dst_hbm, ones_hbm, zeros_hbm, out_hbm, didx, ones, acc):
        cid = lax.axis_index("c")
        sid = lax.axis_index("s")
        wid = cid * NS + sid

        pltpu.sync_copy(ones_hbm, ones)
        pltpu.sync_copy(zeros_hbm, acc.at[pl.ds(sid * RPS, RPS)])
        plsc.subcore_barrier()

        pltpu.sync_copy(dst_hbm.at[wid], didx)

        @pl.loop(0, NCW)
        def _(c):
            pltpu.sync_copy(ones, acc.at[didx.at[c]], add=True)

        plsc.subcore_barrier()
        pltpu.sync_copy(acc.at[pl.ds(sid * RPS, RPS)],
                        out_hbm.at[pl.ds(cid * NP + sid * RPS, RPS)])

    return deg_kernel(dst3d, ones_h, zeros_h)


def _sc_scatter(y, eidx, d):
    zeros_h = jnp.zeros((RPS, d), jnp.float32)

    @functools.partial(
        pl.kernel,
        out_type=jax.ShapeDtypeStruct((NC * NP, d), jnp.float32),
        mesh=_vmesh(),
        scratch_types=[
            pltpu.VMEM((2, CH), jnp.int32),
            pltpu.VMEM((2, CH), jnp.int32),
            pltpu.VMEM((CH, d), jnp.float32),
            pltpu.VMEM((CH, d), jnp.float32),
            pltpu.VMEM_SHARED((NP, d), jnp.float32),
            pltpu.SemaphoreType.DMA,
            pltpu.SemaphoreType.DMA,
            pltpu.SemaphoreType.DMA,
            pltpu.SemaphoreType.DMA,
        ],
        compiler_params=_SC_PARAMS if d != F else None,
    )
    def scatter_kernel(y_hbm, eidx_hbm, zeros_hbm, out_hbm,
                       ib0, ib1, rows0, rows1, acc, sg0, sg1, si0, si1):
        cid = lax.axis_index("c")
        sid = lax.axis_index("s")
        wid = cid * NS + sid
        base = wid * NCW

        pltpu.sync_copy(zeros_hbm, acc.at[pl.ds(sid * RPS, RPS)])
        plsc.subcore_barrier()

        pltpu.sync_copy(eidx_hbm.at[base], ib0)
        pltpu.async_copy(y_hbm.at[ib0.at[0]], rows0, sg0)
        pltpu.async_copy(eidx_hbm.at[base + 1], ib1, si1)

        @pl.loop(0, NCW // 2)
        def _(p):
            c = base + 2 * p
            pltpu.make_async_copy(eidx_hbm.at[base], ib1, si1).wait()
            pltpu.async_copy(y_hbm.at[ib1.at[0]], rows1, sg1)
            pltpu.make_async_copy(y_hbm.at[ib0.at[0]], rows0, sg0).wait()
            pltpu.sync_copy(rows0, acc.at[ib0.at[1]], add=True)
            pltpu.async_copy(eidx_hbm.at[c + 2], ib0, si0)
            pltpu.make_async_copy(eidx_hbm.at[base], ib0, si0).wait()
            pltpu.async_copy(y_hbm.at[ib0.at[0]], rows0, sg0)
            pltpu.make_async_copy(y_hbm.at[ib1.at[0]], rows1, sg1).wait()
            pltpu.sync_copy(rows1, acc.at[ib1.at[1]], add=True)
            pltpu.async_copy(eidx_hbm.at[c + 3], ib1, si1)

        pltpu.make_async_copy(y_hbm.at[ib0.at[0]], rows0, sg0).wait()
        pltpu.sync_copy(rows0, acc.at[ib0.at[1]], add=True)
        pltpu.make_async_copy(eidx_hbm.at[base], ib1, si1).wait()

        plsc.subcore_barrier()
        pltpu.sync_copy(acc.at[pl.ds(sid * RPS, RPS)],
                        out_hbm.at[pl.ds(cid * NP + sid * RPS, RPS)])

    return scatter_kernel(y, eidx, zeros_h)


BM = 1024
NBLK = NP // BM


def _norm_col(dlo_ref, dhi_ref):
    d = dlo_ref[...] + dhi_ref[...]
    return lax.rsqrt(jnp.maximum(d[:, 0:1], 1.0))


def _dot(a, b):
    return jnp.dot(a, b, preferred_element_type=jnp.float32,
                   precision=lax.Precision.HIGHEST)


def _tc_matmul(x, w):
    m, k = x.shape
    dout = w.shape[1]

    def body(x_ref, w_ref, o_ref):
        o_ref[...] = _dot(x_ref[...], w_ref[...])

    return pl.pallas_call(
        body,
        grid=(NBLK,),
        in_specs=[pl.BlockSpec((BM, k), lambda i: (i, 0)),
                  pl.BlockSpec((k, dout), lambda i: (0, 0))],
        out_specs=pl.BlockSpec((BM, dout), lambda i: (i, 0)),
        out_shape=jax.ShapeDtypeStruct((m, dout), jnp.float32),
    )(x, w)


def _deg_specs():
    return [pl.BlockSpec((BM, DEGW), lambda i: (i, 0)),
            pl.BlockSpec((BM, DEGW), lambda i: (i + NBLK, 0))]


def _tc_scale(degp, z):
    def body(dlo, dhi, z_ref, o_ref):
        o_ref[...] = z_ref[...] * _norm_col(dlo, dhi)

    return pl.pallas_call(
        body,
        grid=(NBLK,),
        in_specs=_deg_specs() + [pl.BlockSpec((BM, F), lambda i: (i, 0))],
        out_specs=pl.BlockSpec((BM, F), lambda i: (i, 0)),
        out_shape=jax.ShapeDtypeStruct((N, F), jnp.float32),
    )(degp, degp, z)


def _tc_mid(degp, p, b, w):
    dout = w.shape[1]

    def body(dlo, dhi, plo, phi, b_ref, w_ref, o_ref):
        n = _norm_col(dlo, dhi)
        t = jax.nn.relu(n * (plo[...] + phi[...]) + b_ref[...])
        o_ref[...] = _dot(n * t, w_ref[...])

    return pl.pallas_call(
        body,
        grid=(NBLK,),
        in_specs=_deg_specs() + [
            pl.BlockSpec((BM, F), lambda i: (i, 0)),
            pl.BlockSpec((BM, F), lambda i: (i + NBLK, 0)),
            pl.BlockSpec((1, F), lambda i: (0, 0)),
            pl.BlockSpec((F, dout), lambda i: (0, 0))],
        out_specs=pl.BlockSpec((BM, dout), lambda i: (i, 0)),
        out_shape=jax.ShapeDtypeStruct((N, dout), jnp.float32),
    )(degp, degp, p, p, b, w)


def _tc_final(degp, p, b):
    def body(dlo, dhi, plo, phi, b_ref, o_ref):
        n = _norm_col(dlo, dhi)
        o_ref[...] = (n * (plo[...] + phi[...]) + b_ref[...])[:, :NCLS]

    return pl.pallas_call(
        body,
        grid=(NBLK,),
        in_specs=_deg_specs() + [
            pl.BlockSpec((BM, DPAD), lambda i: (i, 0)),
            pl.BlockSpec((BM, DPAD), lambda i: (i + NBLK, 0)),
            pl.BlockSpec((1, DPAD), lambda i: (0, 0))],
        out_specs=pl.BlockSpec((BM, NCLS), lambda i: (i, 0)),
        out_shape=jax.ShapeDtypeStruct((N, NCLS), jnp.float32),
    )(degp, degp, p, p, b)


def kernel(edge_index, features, W0, b0, W1, b1, W2, b2):
    ei = edge_index.astype(jnp.int32)
    r = jnp.arange(EPAD, dtype=jnp.int32)
    pad = jnp.stack([r % N, N + r % (NP - N)])
    full = jnp.concatenate([ei, pad], axis=1)
    eidx = full.reshape(2, NCHT, CH).transpose(1, 0, 2)
    eidx = jnp.pad(eidx, ((0, 1), (0, 0), (0, 0)))
    dst3d = full[1].reshape(NW, NCW, CH)

    degp = _sc_degree(dst3d)
    z0 = _tc_matmul(features, W0)
    y0 = _tc_scale(degp, z0)

    p0 = _sc_scatter(y0, eidx, F)
    y1 = _tc_mid(degp, p0, b0.reshape(1, F), W1)

    p1 = _sc_scatter(y1, eidx, F)
    w2p = jnp.pad(W2, ((0, 0), (0, DPAD - NCLS)))
    y2 = _tc_mid(degp, p1, b1.reshape(1, F), w2p)

    p2 = _sc_scatter(y2, eidx, DPAD)
    return _tc_final(degp, p2, jnp.pad(b2, (0, DPAD - NCLS)).reshape(1, DPAD))


# speedup vs baseline: 1.0621x; 1.0621x over previous
import functools

import jax
import jax.numpy as jnp
from jax import lax
from jax.experimental import pallas as pl
from jax.experimental.pallas import tpu as pltpu
from jax.experimental.pallas import tpu_sc as plsc

N = 10000
NP = 10240
E = 320000
F = 128
NCLS = 40
DPAD = 48
DEGW = 16

NC, NS = 2, 16
NW = NC * NS
CH = 128
NCW = 79
NCHT = NW * NCW
EPAD = NCHT * CH - E
RPS = NP // NS


def _vmesh():
    return plsc.VectorSubcoreMesh(core_axis_name="c", subcore_axis_name="s")


_SC_PARAMS = pltpu.CompilerParams(use_tc_tiling_on_sc=False)


def _sc_degree(dst3d):
    ones_h = jnp.ones((CH, DEGW), jnp.float32)
    zeros_h = jnp.zeros((RPS, DEGW), jnp.float32)

    @functools.partial(
        pl.kernel,
        out_type=jax.ShapeDtypeStruct((NC * NP, DEGW), jnp.float32),
        mesh=_vmesh(),
        scratch_types=[
            pltpu.VMEM((NCW, CH), jnp.int32),
            pltpu.VMEM((CH, DEGW), jnp.float32),
            pltpu.VMEM_SHARED((NP, DEGW), jnp.float32),
        ],
        compiler_params=_SC_PARAMS,
    )
    def deg_kernel(dst_hbm, ones_hbm, zeros_hbm, out_hbm, didx, ones, acc):
        cid = lax.axis_index("c")
        sid = lax.axis_index("s")
        wid = cid * NS + sid

        pltpu.sync_copy(ones_hbm, ones)
        pltpu.sync_copy(zeros_hbm, acc.at[pl.ds(sid * RPS, RPS)])
        plsc.subcore_barrier()

        pltpu.sync_copy(dst_hbm.at[wid], didx)

        @pl.loop(0, NCW)
        def _(c):
            pltpu.sync_copy(ones, acc.at[didx.at[c]], add=True)

        plsc.subcore_barrier()
        pltpu.sync_copy(acc.at[pl.ds(sid * RPS, RPS)],
                        out_hbm.at[pl.ds(cid * NP + sid * RPS, RPS)])

    return deg_kernel(dst3d, ones_h, zeros_h)


def _sc_scatter(y, eidx, d):
    zeros_h = jnp.zeros((RPS, d), jnp.float32)

    @functools.partial(
        pl.kernel,
        out_type=jax.ShapeDtypeStruct((NC * NP, d), jnp.float32),
        mesh=_vmesh(),
        scratch_types=[
            pltpu.VMEM((2, CH), jnp.int32),
            pltpu.VMEM((2, CH), jnp.int32),
            pltpu.VMEM((CH, d), jnp.float32),
            pltpu.VMEM((CH, d), jnp.float32),
            pltpu.VMEM_SHARED((NP, d), jnp.float32),
            pltpu.SemaphoreType.DMA,
            pltpu.SemaphoreType.DMA,
            pltpu.SemaphoreType.DMA,
            pltpu.SemaphoreType.DMA,
        ],
        compiler_params=_SC_PARAMS,
    )
    def scatter_kernel(y_hbm, eidx_hbm, zeros_hbm, out_hbm,
                       ib0, ib1, rows0, rows1, acc, sg0, sg1, si0, si1):
        cid = lax.axis_index("c")
        sid = lax.axis_index("s")
        wid = cid * NS + sid
        base = wid * NCW

        pltpu.sync_copy(zeros_hbm, acc.at[pl.ds(sid * RPS, RPS)])
        plsc.subcore_barrier()

        pltpu.sync_copy(eidx_hbm.at[base], ib0)
        pltpu.async_copy(y_hbm.at[ib0.at[0]], rows0, sg0)
        pltpu.async_copy(eidx_hbm.at[base + 1], ib1, si1)

        @pl.loop(0, NCW // 2)
        def _(p):
            c = base + 2 * p
            pltpu.make_async_copy(eidx_hbm.at[base], ib1, si1).wait()
            pltpu.async_copy(y_hbm.at[ib1.at[0]], rows1, sg1)
            pltpu.make_async_copy(y_hbm.at[ib0.at[0]], rows0, sg0).wait()
            pltpu.sync_copy(rows0, acc.at[ib0.at[1]], add=True)
            pltpu.async_copy(eidx_hbm.at[c + 2], ib0, si0)
            pltpu.make_async_copy(eidx_hbm.at[base], ib0, si0).wait()
            pltpu.async_copy(y_hbm.at[ib0.at[0]], rows0, sg0)
            pltpu.make_async_copy(y_hbm.at[ib1.at[0]], rows1, sg1).wait()
            pltpu.sync_copy(rows1, acc.at[ib1.at[1]], add=True)
            pltpu.async_copy(eidx_hbm.at[c + 3], ib1, si1)

        pltpu.make_async_copy(y_hbm.at[ib0.at[0]], rows0, sg0).wait()
        pltpu.sync_copy(rows0, acc.at[ib0.at[1]], add=True)
        pltpu.make_async_copy(eidx_hbm.at[base], ib1, si1).wait()

        plsc.subcore_barrier()
        pltpu.sync_copy(acc.at[pl.ds(sid * RPS, RPS)],
                        out_hbm.at[pl.ds(cid * NP + sid * RPS, RPS)])

    return scatter_kernel(y, eidx, zeros_h)


BM = 2048
NBLK = NP // BM


def _norm_col(dlo_ref, dhi_ref):
    d = dlo_ref[...] + dhi_ref[...]
    return lax.rsqrt(jnp.maximum(d[:, 0:1], 1.0))


def _dot(a, b):
    return jnp.dot(a, b, preferred_element_type=jnp.float32,
                   precision=lax.Precision.HIGHEST)


def _tc_matmul(x, w):
    m, k = x.shape
    dout = w.shape[1]

    def body(x_ref, w_ref, o_ref):
        o_ref[...] = _dot(x_ref[...], w_ref[...])

    return pl.pallas_call(
        body,
        grid=(NBLK,),
        in_specs=[pl.BlockSpec((BM, k), lambda i: (i, 0)),
                  pl.BlockSpec((k, dout), lambda i: (0, 0))],
        out_specs=pl.BlockSpec((BM, dout), lambda i: (i, 0)),
        out_shape=jax.ShapeDtypeStruct((m, dout), jnp.float32),
    )(x, w)


def _deg_specs():
    return [pl.BlockSpec((BM, DEGW), lambda i: (i, 0)),
            pl.BlockSpec((BM, DEGW), lambda i: (i + NBLK, 0))]


def _tc_scale(degp, z):
    def body(dlo, dhi, z_ref, o_ref):
        o_ref[...] = z_ref[...] * _norm_col(dlo, dhi)

    return pl.pallas_call(
        body,
        grid=(NBLK,),
        in_specs=_deg_specs() + [pl.BlockSpec((BM, F), lambda i: (i, 0))],
        out_specs=pl.BlockSpec((BM, F), lambda i: (i, 0)),
        out_shape=jax.ShapeDtypeStruct((N, F), jnp.float32),
    )(degp, degp, z)


def _tc_mid(degp, p, b, w):
    dout = w.shape[1]

    def body(dlo, dhi, plo, phi, b_ref, w_ref, o_ref):
        n = _norm_col(dlo, dhi)
        t = jax.nn.relu(n * (plo[...] + phi[...]) + b_ref[...])
        o_ref[...] = _dot(n * t, w_ref[...])

    return pl.pallas_call(
        body,
        grid=(NBLK,),
        in_specs=_deg_specs() + [
            pl.BlockSpec((BM, F), lambda i: (i, 0)),
            pl.BlockSpec((BM, F), lambda i: (i + NBLK, 0)),
            pl.BlockSpec((1, F), lambda i: (0, 0)),
            pl.BlockSpec((F, dout), lambda i: (0, 0))],
        out_specs=pl.BlockSpec((BM, dout), lambda i: (i, 0)),
        out_shape=jax.ShapeDtypeStruct((N, dout), jnp.float32),
    )(degp, degp, p, p, b, w)


def _tc_final(degp, p, b):
    def body(dlo, dhi, plo, phi, b_ref, o_ref):
        n = _norm_col(dlo, dhi)
        o_ref[...] = (n * (plo[...] + phi[...]) + b_ref[...])[:, :NCLS]

    return pl.pallas_call(
        body,
        grid=(NBLK,),
        in_specs=_deg_specs() + [
            pl.BlockSpec((BM, DPAD), lambda i: (i, 0)),
            pl.BlockSpec((BM, DPAD), lambda i: (i + NBLK, 0)),
            pl.BlockSpec((1, DPAD), lambda i: (0, 0))],
        out_specs=pl.BlockSpec((BM, NCLS), lambda i: (i, 0)),
        out_shape=jax.ShapeDtypeStruct((N, NCLS), jnp.float32),
    )(degp, degp, p, p, b)


def kernel(edge_index, features, W0, b0, W1, b1, W2, b2):
    ei = edge_index.astype(jnp.int32)
    r = jnp.arange(EPAD, dtype=jnp.int32)
    pad = jnp.stack([r % N, N + r % (NP - N)])
    full = jnp.concatenate([ei, pad], axis=1)
    eidx = full.reshape(2, NCHT, CH).transpose(1, 0, 2)
    eidx = jnp.pad(eidx, ((0, 1), (0, 0), (0, 0)))
    dst3d = full[1].reshape(NW, NCW, CH)

    degp = _sc_degree(dst3d)
    z0 = _tc_matmul(features, W0)
    y0 = _tc_scale(degp, z0)

    p0 = _sc_scatter(y0, eidx, F)
    y1 = _tc_mid(degp, p0, b0.reshape(1, F), W1)

    p1 = _sc_scatter(y1, eidx, F)
    w2p = jnp.pad(W2, ((0, 0), (0, DPAD - NCLS)))
    y2 = _tc_mid(degp, p1, b1.reshape(1, F), w2p)

    p2 = _sc_scatter(y2, eidx, DPAD)
    return _tc_final(degp, p2, jnp.pad(b2, (0, DPAD - NCLS)).reshape(1, DPAD))


# speedup vs baseline: 1.0748x; 1.0120x over previous
import functools

import jax
import jax.numpy as jnp
from jax import lax
from jax.experimental import pallas as pl
from jax.experimental.pallas import tpu as pltpu
from jax.experimental.pallas import tpu_sc as plsc

N = 10000
NP = 10240
E = 320000
F = 128
NCLS = 40
DPAD = 48
DEGW = 16

NC, NS = 2, 16
NW = NC * NS
CH = 128
NCW = 79
NCHT = NW * NCW
EPAD = NCHT * CH - E
RPS = NP // NS


def _vmesh():
    return plsc.VectorSubcoreMesh(core_axis_name="c", subcore_axis_name="s")


_SC_PARAMS = pltpu.CompilerParams(use_tc_tiling_on_sc=False)


def _sc_degree(dst3d):
    ones_h = jnp.ones((CH, DEGW), jnp.float32)
    zeros_h = jnp.zeros((RPS, DEGW), jnp.float32)

    @functools.partial(
        pl.kernel,
        out_type=jax.ShapeDtypeStruct((NC * NP, DEGW), jnp.float32),
        mesh=_vmesh(),
        scratch_types=[
            pltpu.VMEM((NCW, CH), jnp.int32),
            pltpu.VMEM((CH, DEGW), jnp.float32),
            pltpu.VMEM_SHARED((NP, DEGW), jnp.float32),
        ],
        compiler_params=_SC_PARAMS,
    )
    def deg_kernel(dst_hbm, ones_hbm, zeros_hbm, out_hbm, didx, ones, acc):
        cid = lax.axis_index("c")
        sid = lax.axis_index("s")
        wid = cid * NS + sid

        pltpu.sync_copy(ones_hbm, ones)
        pltpu.sync_copy(zeros_hbm, acc.at[pl.ds(sid * RPS, RPS)])
        plsc.subcore_barrier()

        pltpu.sync_copy(dst_hbm.at[wid], didx)

        @pl.loop(0, NCW)
        def _(c):
            pltpu.sync_copy(ones, acc.at[didx.at[c]], add=True)

        plsc.subcore_barrier()
        pltpu.sync_copy(acc.at[pl.ds(sid * RPS, RPS)],
                        out_hbm.at[pl.ds(cid * NP + sid * RPS, RPS)])

    return deg_kernel(dst3d, ones_h, zeros_h)


def _sc_scatter(y, eidx, d):
    zeros_h = jnp.zeros((RPS, d), jnp.float32)

    @functools.partial(
        pl.kernel,
        out_type=jax.ShapeDtypeStruct((NC * NP, d), jnp.float32),
        mesh=_vmesh(),
        scratch_types=[
            pltpu.VMEM((2, CH), jnp.int32),
            pltpu.VMEM((2, CH), jnp.int32),
            pltpu.VMEM((CH, d), jnp.float32),
            pltpu.VMEM((CH, d), jnp.float32),
            pltpu.VMEM_SHARED((NP, d), jnp.float32),
            pltpu.SemaphoreType.DMA,
            pltpu.SemaphoreType.DMA,
            pltpu.SemaphoreType.DMA,
            pltpu.SemaphoreType.DMA,
        ],
        compiler_params=_SC_PARAMS,
    )
    def scatter_kernel(y_hbm, eidx_hbm, zeros_hbm, out_hbm,
                       ib0, ib1, rows0, rows1, acc, sg0, sg1, si0, si1):
        cid = lax.axis_index("c")
        sid = lax.axis_index("s")
        wid = cid * NS + sid
        base = wid * NCW

        pltpu.sync_copy(zeros_hbm, acc.at[pl.ds(sid * RPS, RPS)])
        plsc.subcore_barrier()

        pltpu.sync_copy(eidx_hbm.at[base], ib0)
        pltpu.async_copy(y_hbm.at[ib0.at[0]], rows0, sg0)
        pltpu.async_copy(eidx_hbm.at[base + 1], ib1, si1)

        @pl.loop(0, NCW // 2)
        def _(p):
            c = base + 2 * p
            pltpu.make_async_copy(eidx_hbm.at[base], ib1, si1).wait()
            pltpu.async_copy(y_hbm.at[ib1.at[0]], rows1, sg1)
            pltpu.make_async_copy(y_hbm.at[ib0.at[0]], rows0, sg0).wait()
            pltpu.sync_copy(rows0, acc.at[ib0.at[1]], add=True)
            pltpu.async_copy(eidx_hbm.at[c + 2], ib0, si0)
            pltpu.make_async_copy(eidx_hbm.at[base], ib0, si0).wait()
            pltpu.async_copy(y_hbm.at[ib0.at[0]], rows0, sg0)
            pltpu.make_async_copy(y_hbm.at[ib1.at[0]], rows1, sg1).wait()
            pltpu.sync_copy(rows1, acc.at[ib1.at[1]], add=True)
            pltpu.async_copy(eidx_hbm.at[c + 3], ib1, si1)

        pltpu.make_async_copy(y_hbm.at[ib0.at[0]], rows0, sg0).wait()
        pltpu.sync_copy(rows0, acc.at[ib0.at[1]], add=True)
        pltpu.make_async_copy(eidx_hbm.at[base], ib1, si1).wait()

        plsc.subcore_barrier()
        pltpu.sync_copy(acc.at[pl.ds(sid * RPS, RPS)],
                        out_hbm.at[pl.ds(cid * NP + sid * RPS, RPS)])

    return scatter_kernel(y, eidx, zeros_h)


BM = 2048
NBLK = NP // BM


def _norm_col(dlo_ref, dhi_ref):
    d = dlo_ref[...] + dhi_ref[...]
    return lax.rsqrt(jnp.maximum(d[:, 0:1], 1.0))


def _dot(a, b):
    return jnp.dot(a, b, preferred_element_type=jnp.float32)


def _tc_matmul(x, w):
    m, k = x.shape
    dout = w.shape[1]

    def body(x_ref, w_ref, o_ref):
        o_ref[...] = _dot(x_ref[...], w_ref[...])

    return pl.pallas_call(
        body,
        grid=(NBLK,),
        in_specs=[pl.BlockSpec((BM, k), lambda i: (i, 0)),
                  pl.BlockSpec((k, dout), lambda i: (0, 0))],
        out_specs=pl.BlockSpec((BM, dout), lambda i: (i, 0)),
        out_shape=jax.ShapeDtypeStruct((m, dout), jnp.float32),
    )(x, w)


def _deg_specs():
    return [pl.BlockSpec((BM, DEGW), lambda i: (i, 0)),
            pl.BlockSpec((BM, DEGW), lambda i: (i + NBLK, 0))]


def _tc_scale(degp, z):
    def body(dlo, dhi, z_ref, o_ref):
        o_ref[...] = z_ref[...] * _norm_col(dlo, dhi)

    return pl.pallas_call(
        body,
        grid=(NBLK,),
        in_specs=_deg_specs() + [pl.BlockSpec((BM, F), lambda i: (i, 0))],
        out_specs=pl.BlockSpec((BM, F), lambda i: (i, 0)),
        out_shape=jax.ShapeDtypeStruct((N, F), jnp.float32),
    )(degp, degp, z)


def _tc_mid(degp, p, b, w):
    dout = w.shape[1]

    def body(dlo, dhi, plo, phi, b_ref, w_ref, o_ref):
        n = _norm_col(dlo, dhi)
        t = jax.nn.relu(n * (plo[...] + phi[...]) + b_ref[...])
        o_ref[...] = _dot(n * t, w_ref[...])

    return pl.pallas_call(
        body,
        grid=(NBLK,),
        in_specs=_deg_specs() + [
            pl.BlockSpec((BM, F), lambda i: (i, 0)),
            pl.BlockSpec((BM, F), lambda i: (i + NBLK, 0)),
            pl.BlockSpec((1, F), lambda i: (0, 0)),
            pl.BlockSpec((F, dout), lambda i: (0, 0))],
        out_specs=pl.BlockSpec((BM, dout), lambda i: (i, 0)),
        out_shape=jax.ShapeDtypeStruct((N, dout), jnp.float32),
    )(degp, degp, p, p, b, w)


def _tc_final(degp, p, b):
    def body(dlo, dhi, plo, phi, b_ref, o_ref):
        n = _norm_col(dlo, dhi)
        o_ref[...] = (n * (plo[...] + phi[...]) + b_ref[...])[:, :NCLS]

    return pl.pallas_call(
        body,
        grid=(NBLK,),
        in_specs=_deg_specs() + [
            pl.BlockSpec((BM, DPAD), lambda i: (i, 0)),
            pl.BlockSpec((BM, DPAD), lambda i: (i + NBLK, 0)),
            pl.BlockSpec((1, DPAD), lambda i: (0, 0))],
        out_specs=pl.BlockSpec((BM, NCLS), lambda i: (i, 0)),
        out_shape=jax.ShapeDtypeStruct((N, NCLS), jnp.float32),
    )(degp, degp, p, p, b)


def kernel(edge_index, features, W0, b0, W1, b1, W2, b2):
    ei = edge_index.astype(jnp.int32)
    r = jnp.arange(EPAD, dtype=jnp.int32)
    pad = jnp.stack([r % N, N + r % (NP - N)])
    full = jnp.concatenate([ei, pad], axis=1)
    eidx = full.reshape(2, NCHT, CH).transpose(1, 0, 2)
    eidx = jnp.pad(eidx, ((0, 1), (0, 0), (0, 0)))
    dst3d = full[1].reshape(NW, NCW, CH)

    degp = _sc_degree(dst3d)
    z0 = _tc_matmul(features, W0)
    y0 = _tc_scale(degp, z0)

    p0 = _sc_scatter(y0, eidx, F)
    y1 = _tc_mid(degp, p0, b0.reshape(1, F), W1)

    p1 = _sc_scatter(y1, eidx, F)
    w2p = jnp.pad(W2, ((0, 0), (0, DPAD - NCLS)))
    y2 = _tc_mid(degp, p1, b1.reshape(1, F), w2p)

    p2 = _sc_scatter(y2, eidx, DPAD)
    return _tc_final(degp, p2, jnp.pad(b2, (0, DPAD - NCLS)).reshape(1, DPAD))


# speedup vs baseline: 1.0785x; 1.0034x over previous
import functools

import jax
import jax.numpy as jnp
from jax import lax
from jax.experimental import pallas as pl
from jax.experimental.pallas import tpu as pltpu
from jax.experimental.pallas import tpu_sc as plsc

N = 10000
NP = 10240
E = 320000
F = 128
NCLS = 40
DPAD = 48
DEGW = 16

NC, NS = 2, 16
NW = NC * NS
CH = 128
NCW = 79
NCHT = NW * NCW
EPAD = NCHT * CH - E
RPS = NP // NS


def _vmesh():
    return plsc.VectorSubcoreMesh(core_axis_name="c", subcore_axis_name="s")


_SC_PARAMS = pltpu.CompilerParams(use_tc_tiling_on_sc=False)


def _sc_degree(dst3d):
    ones_h = jnp.ones((CH, DEGW), jnp.float32)
    zeros_h = jnp.zeros((RPS, DEGW), jnp.float32)

    @functools.partial(
        pl.kernel,
        out_type=jax.ShapeDtypeStruct((NC * NP, DEGW), jnp.float32),
        mesh=_vmesh(),
        scratch_types=[
            pltpu.VMEM((NCW, CH), jnp.int32),
            pltpu.VMEM((CH, DEGW), jnp.float32),
            pltpu.VMEM_SHARED((NP, DEGW), jnp.float32),
        ],
        compiler_params=_SC_PARAMS,
    )
    def deg_kernel(dst_hbm, ones_hbm, zeros_hbm, out_hbm, didx, ones, acc):
        cid = lax.axis_index("c")
        sid = lax.axis_index("s")
        wid = cid * NS + sid

        pltpu.sync_copy(ones_hbm, ones)
        pltpu.sync_copy(zeros_hbm, acc.at[pl.ds(sid * RPS, RPS)])
        plsc.subcore_barrier()

        pltpu.sync_copy(dst_hbm.at[wid], didx)

        @pl.loop(0, NCW)
        def _(c):
            pltpu.sync_copy(ones, acc.at[didx.at[c]], add=True)

        plsc.subcore_barrier()
        pltpu.sync_copy(acc.at[pl.ds(sid * RPS, RPS)],
                        out_hbm.at[pl.ds(cid * NP + sid * RPS, RPS)])

    return deg_kernel(dst3d, ones_h, zeros_h)


def _sc_scatter(y, eidx, d):
    zeros_h = jnp.zeros((RPS, d), jnp.float32)

    @functools.partial(
        pl.kernel,
        out_type=jax.ShapeDtypeStruct((NC * NP, d), jnp.float32),
        mesh=_vmesh(),
        scratch_types=[
            pltpu.VMEM((2, CH), jnp.int32),
            pltpu.VMEM((2, CH), jnp.int32),
            pltpu.VMEM((CH, d), jnp.float32),
            pltpu.VMEM((CH, d), jnp.float32),
            pltpu.VMEM_SHARED((NP, d), jnp.float32),
            pltpu.SemaphoreType.DMA,
            pltpu.SemaphoreType.DMA,
            pltpu.SemaphoreType.DMA,
            pltpu.SemaphoreType.DMA,
        ],
        compiler_params=_SC_PARAMS,
    )
    def scatter_kernel(y_hbm, eidx_hbm, zeros_hbm, out_hbm,
                       ib0, ib1, rows0, rows1, acc, sg0, sg1, si0, si1):
        cid = lax.axis_index("c")
        sid = lax.axis_index("s")
        wid = cid * NS + sid
        base = wid * NCW

        pltpu.sync_copy(zeros_hbm, acc.at[pl.ds(sid * RPS, RPS)])
        plsc.subcore_barrier()

        pltpu.sync_copy(eidx_hbm.at[base], ib0)
        pltpu.async_copy(y_hbm.at[ib0.at[0]], rows0, sg0)
        pltpu.async_copy(eidx_hbm.at[base + 1], ib1, si1)

        @pl.loop(0, NCW // 2)
        def _(p):
            c = base + 2 * p
            pltpu.make_async_copy(eidx_hbm.at[base], ib1, si1).wait()
            pltpu.async_copy(y_hbm.at[ib1.at[0]], rows1, sg1)
            pltpu.make_async_copy(y_hbm.at[ib0.at[0]], rows0, sg0).wait()
            pltpu.sync_copy(rows0, acc.at[ib0.at[1]], add=True)
            pltpu.async_copy(eidx_hbm.at[c + 2], ib0, si0)
            pltpu.make_async_copy(eidx_hbm.at[base], ib0, si0).wait()
            pltpu.async_copy(y_hbm.at[ib0.at[0]], rows0, sg0)
            pltpu.make_async_copy(y_hbm.at[ib1.at[0]], rows1, sg1).wait()
            pltpu.sync_copy(rows1, acc.at[ib1.at[1]], add=True)
            pltpu.async_copy(eidx_hbm.at[c + 3], ib1, si1)

        pltpu.make_async_copy(y_hbm.at[ib0.at[0]], rows0, sg0).wait()
        pltpu.sync_copy(rows0, acc.at[ib0.at[1]], add=True)
        pltpu.make_async_copy(eidx_hbm.at[base], ib1, si1).wait()

        plsc.subcore_barrier()
        pltpu.sync_copy(acc.at[pl.ds(sid * RPS, RPS)],
                        out_hbm.at[pl.ds(cid * NP + sid * RPS, RPS)])

    return scatter_kernel(y, eidx, zeros_h)


BM = 2560
NBLK = NP // BM


def _norm_col(dlo_ref, dhi_ref):
    d = dlo_ref[...] + dhi_ref[...]
    return lax.rsqrt(jnp.maximum(d[:, 0:1], 1.0))


def _dot(a, b):
    return jnp.dot(a, b, preferred_element_type=jnp.float32)


def _tc_matmul(x, w):
    m, k = x.shape
    dout = w.shape[1]

    def body(x_ref, w_ref, o_ref):
        o_ref[...] = _dot(x_ref[...], w_ref[...])

    return pl.pallas_call(
        body,
        grid=(NBLK,),
        in_specs=[pl.BlockSpec((BM, k), lambda i: (i, 0)),
                  pl.BlockSpec((k, dout), lambda i: (0, 0))],
        out_specs=pl.BlockSpec((BM, dout), lambda i: (i, 0)),
        out_shape=jax.ShapeDtypeStruct((m, dout), jnp.float32),
    )(x, w)


def _deg_specs():
    return [pl.BlockSpec((BM, DEGW), lambda i: (i, 0)),
            pl.BlockSpec((BM, DEGW), lambda i: (i + NBLK, 0))]


def _tc_scale(degp, z):
    def body(dlo, dhi, z_ref, o_ref):
        o_ref[...] = z_ref[...] * _norm_col(dlo, dhi)

    return pl.pallas_call(
        body,
        grid=(NBLK,),
        in_specs=_deg_specs() + [pl.BlockSpec((BM, F), lambda i: (i, 0))],
        out_specs=pl.BlockSpec((BM, F), lambda i: (i, 0)),
        out_shape=jax.ShapeDtypeStruct((N, F), jnp.float32),
    )(degp, degp, z)


def _tc_mid(degp, p, b, w):
    dout = w.shape[1]

    def body(dlo, dhi, plo, phi, b_ref, w_ref, o_ref):
        n = _norm_col(dlo, dhi)
        t = jax.nn.relu(n * (plo[...] + phi[...]) + b_ref[...])
        o_ref[...] = _dot(n * t, w_ref[...])

    return pl.pallas_call(
        body,
        grid=(NBLK,),
        in_specs=_deg_specs() + [
            pl.BlockSpec((BM, F), lambda i: (i, 0)),
            pl.BlockSpec((BM, F), lambda i: (i + NBLK, 0)),
            pl.BlockSpec((1, F), lambda i: (0, 0)),
            pl.BlockSpec((F, dout), lambda i: (0, 0))],
        out_specs=pl.BlockSpec((BM, dout), lambda i: (i, 0)),
        out_shape=jax.ShapeDtypeStruct((N, dout), jnp.float32),
    )(degp, degp, p, p, b, w)


def _tc_final(degp, p, b):
    def body(dlo, dhi, plo, phi, b_ref, o_ref):
        n = _norm_col(dlo, dhi)
        o_ref[...] = (n * (plo[...] + phi[...]) + b_ref[...])[:, :NCLS]

    return pl.pallas_call(
        body,
        grid=(NBLK,),
        in_specs=_deg_specs() + [
            pl.BlockSpec((BM, DPAD), lambda i: (i, 0)),
            pl.BlockSpec((BM, DPAD), lambda i: (i + NBLK, 0)),
            pl.BlockSpec((1, DPAD), lambda i: (0, 0))],
        out_specs=pl.BlockSpec((BM, NCLS), lambda i: (i, 0)),
        out_shape=jax.ShapeDtypeStruct((N, NCLS), jnp.float32),
    )(degp, degp, p, p, b)


def kernel(edge_index, features, W0, b0, W1, b1, W2, b2):
    ei = edge_index.astype(jnp.int32)
    r = jnp.arange(EPAD, dtype=jnp.int32)
    pad = jnp.stack([r % N, N + r % (NP - N)])
    full = jnp.concatenate([ei, pad], axis=1)
    eidx = full.reshape(2, NCHT, CH).transpose(1, 0, 2)
    eidx = jnp.pad(eidx, ((0, 1), (0, 0), (0, 0)))
    dst3d = full[1].reshape(NW, NCW, CH)

    degp = _sc_degree(dst3d)
    z0 = _tc_matmul(features, W0)
    y0 = _tc_scale(degp, z0)

    p0 = _sc_scatter(y0, eidx, F)
    y1 = _tc_mid(degp, p0, b0.reshape(1, F), W1)

    p1 = _sc_scatter(y1, eidx, F)
    w2p = jnp.pad(W2, ((0, 0), (0, DPAD - NCLS)))
    y2 = _tc_mid(degp, p1, b1.reshape(1, F), w2p)

    p2 = _sc_scatter(y2, eidx, DPAD)
    return _tc_final(degp, p2, jnp.pad(b2, (0, DPAD - NCLS)).reshape(1, DPAD))
